# baseline (speedup 1.0000x reference)
import functools

import jax
import jax.numpy as jnp
from jax import lax
from jax.experimental import pallas as pl
from jax.experimental.pallas import tpu as pltpu
from jax.experimental.pallas import tpu_sc as plsc

N_NODES = 10000
N_PAD = 10240
F_IN = 128
HID = 64
N_GRAPHS = 16

ROW = 128
CHUNK = 80
NC = 2
NS = 16
LANES = 16


def _mm_body(x_ref, wl_ref, wr_ref, xlr_ref):
    x = x_ref[...]
    xlr_ref[:, :HID] = jnp.dot(x, wl_ref[...],
                               preferred_element_type=jnp.float32)
    xlr_ref[:, HID:] = jnp.dot(x, wr_ref[...],
                               preferred_element_type=jnp.float32)


def _edge_pass(n_edges):
    e_per_tile = n_edges // (NC * NS)
    n_chunks = e_per_tile // CHUNK
    assert e_per_tile * NC * NS == n_edges and n_chunks * CHUNK == e_per_tile
    rows_per_tile = N_PAD // NS
    zrows = 32
    mesh = plsc.VectorSubcoreMesh(core_axis_name="c", subcore_axis_name="s")

    seg_split = 64
    segw = (seg_split + 1) * CHUNK

    @functools.partial(
        pl.kernel,
        out_type=jax.ShapeDtypeStruct((NC, N_PAD, ROW), jnp.float32),
        mesh=mesh,
        scratch_types=[
            pltpu.VMEM((segw,), jnp.int32),
            pltpu.VMEM((CHUNK,), jnp.int32),
            pltpu.VMEM((CHUNK,), jnp.int32),
            pltpu.VMEM((CHUNK,), jnp.int32),
            pltpu.VMEM((CHUNK,), jnp.int32),
            pltpu.VMEM((CHUNK, F_IN), jnp.float32),
            pltpu.VMEM((CHUNK, F_IN), jnp.float32),
            pltpu.VMEM((CHUNK, F_IN), jnp.float32),
            pltpu.VMEM((CHUNK, ROW), jnp.float32),
            pltpu.VMEM((HID,), jnp.float32),
            pltpu.VMEM((LANES * 17,), jnp.float32),
            pltpu.VMEM((CHUNK,), jnp.float32),
            pltpu.VMEM_SHARED((N_PAD, ROW), jnp.float32),
            pltpu.SemaphoreType.DMA,
            pltpu.SemaphoreType.DMA,
            pltpu.SemaphoreType.DMA,
        ],
        compiler_params=pltpu.CompilerParams(needs_layout_passes=False),
    )
    def k(xlr, pk3, att, out,
          idx_pk, sr0, sr1, dr0, dr1, rs0, rs1, rd, obuf, attv, qbuf,
          wbuf, acc, sgs0, sgs1, sgd):
        c = lax.axis_index("c")
        s = lax.axis_index("s")
        tid = c * NS + s

        def zrow(i, carry):
            for j in range(ROW // LANES):
                obuf[i, pl.ds(LANES * j, LANES)] = jnp.zeros((LANES,),
                                                             jnp.float32)
            return carry
        lax.fori_loop(0, CHUNK, zrow, 0)
        for t in range(rows_per_tile // CHUNK):
            row0 = s * rows_per_tile + t * CHUNK
            pltpu.sync_copy(obuf, acc.at[pl.ds(row0, CHUNK)])

        pltpu.sync_copy(att, attv)
        pltpu.sync_copy(pk3.at[tid, 0], idx_pk)
        plsc.subcore_barrier()

        iota = lax.iota(jnp.int32, LANES)
        zero_i = jnp.zeros((LANES,), jnp.int32)
        att_vs = [attv[pl.ds(LANES * kk, LANES)] for kk in range(HID // LANES)]

        def unpack(lch, idx_sr, idx_dr):
            for g in range(CHUNK // LANES):
                pk = idx_pk[pl.ds(lch * CHUNK + g * LANES, LANES)]
                idx_sr[pl.ds(g * LANES, LANES)] = pk & 0xFFFF
                idx_dr[pl.ds(g * LANES, LANES)] = pk >> 16

        iota17 = iota * 17

        def phase1(rs):
            def group_body(g, carry):
                for e16 in range(LANES):
                    e = g * LANES + e16
                    ps = []
                    for kk in range(HID // LANES):
                        l = rs[e, pl.ds(LANES * kk, LANES)]
                        r = rd[e, pl.ds(HID + LANES * kk, LANES)]
                        v = l + r
                        v = jnp.maximum(v, 0.2 * v)
                        ps.append(v * att_vs[kk])
                    qbuf[pl.ds(e16 * 17, LANES)] = (ps[0] + ps[1]) + (
                        ps[2] + ps[3])
                cols = [plsc.load_gather(qbuf, [iota17 + j])
                        for j in range(LANES)]
                while len(cols) > 1:
                    cols = [cols[i] + cols[i + 1]
                            for i in range(0, len(cols), 2)]
                wbuf[pl.ds(g * LANES, LANES)] = jnp.exp(cols[0])
                return carry
            lax.fori_loop(0, CHUNK // LANES, group_body, 0)

        def phase2(rs):
            def edge_body(e, carry):
                wv = plsc.load_gather(wbuf, [zero_i + e])
                for k2 in range(HID // LANES):
                    obuf[e, pl.ds(LANES * k2, LANES)] = (
                        wv * rs[e, pl.ds(LANES * k2, LANES)])
                obuf[e, pl.ds(HID, LANES)] = jnp.where(iota == 0, wv, 0.0)
                return carry
            lax.fori_loop(0, CHUNK, edge_body, 0, unroll=8)

        def fire_rd(idx_dr):
            pltpu.async_copy(xlr.at[idx_dr], rd, sgd)

        def wait_rd(idx_dr):
            pltpu.make_async_copy(xlr.at[idx_dr], rd, sgd).wait()

        def fire_rs(idx_sr, rs, sem):
            pltpu.async_copy(xlr.at[idx_sr], rs, sem)

        def wait_rs(idx_sr, rs, sem):
            pltpu.make_async_copy(xlr.at[idx_sr], rs, sem).wait()

        def section(lch_next, sr_cur, dr_cur, rs_cur, sem_cur,
                    sr_nxt, dr_nxt, rs_nxt, sem_nxt, last=False):
            if not last:
                unpack(lch_next, sr_nxt, dr_nxt)
            wait_rd(dr_cur)
            wait_rs(sr_cur, rs_cur, sem_cur)
            phase1(rs_cur)
            if not last:
                fire_rd(dr_nxt)
            phase2(rs_cur)
            if not last:
                fire_rs(sr_nxt, rs_nxt, sem_nxt)
            pltpu.sync_copy(obuf, acc.at[dr_cur], add=True)

        def pair_body(i, carry):
            section(2 * i + 1, sr0, dr0, rs0, sgs0, sr1, dr1, rs1, sgs1)
            section(2 * i + 2, sr1, dr1, rs1, sgs1, sr0, dr0, rs0, sgs0)
            return carry

        unpack(0, sr0, dr0)
        fire_rd(dr0)
        fire_rs(sr0, rs0, sgs0)
        lax.fori_loop(0, seg_split // 2, pair_body, 0)

        pltpu.sync_copy(pk3.at[tid, 1], idx_pk)
        lax.fori_loop(0, (n_chunks - seg_split - 1) // 2, pair_body, 0)
        section(0, sr0, dr0, rs0, sgs0, sr1, dr1, rs1, sgs1, last=True)

        plsc.subcore_barrier()
        for t in range(rows_per_tile // CHUNK):
            row0 = s * rows_per_tile + t * CHUNK
            pltpu.sync_copy(acc.at[pl.ds(row0, CHUNK)], obuf)
            pltpu.sync_copy(obuf, out.at[c, pl.ds(row0, CHUNK)])

    return k


def _post_body(parts_ref, bconv_ref, lnw_ref, lnb_ref, batch_ref,
               wout_ref, bout_ref, y_ref):
    accp = parts_ref[0, :N_NODES] + parts_ref[1, :N_NODES]
    num = accp[:, :HID]
    den = accp[:, HID:HID + 1]
    h = jnp.maximum(num / (den + 1e-16) + bconv_ref[...], 0.0)
    mu = jnp.mean(h)
    var = jnp.mean((h - mu) ** 2)
    hn = (h - mu) / (jnp.sqrt(var) + 1e-5) * lnw_ref[...] + lnb_ref[...]
    onehot = (batch_ref[...] == lax.broadcasted_iota(
        jnp.int32, (N_NODES, N_GRAPHS), 1)).astype(jnp.float32)
    sums = lax.dot_general(onehot, hn, (((0,), (0,)), ((), ())),
                           preferred_element_type=jnp.float32)
    cnts = jnp.sum(onehot, axis=0)
    pooled = sums / jnp.maximum(cnts, 1.0)[:, None]
    y = jnp.dot(pooled, wout_ref[...], preferred_element_type=jnp.float32)
    y_ref[...] = jax.nn.sigmoid(y + bout_ref[...])


def kernel(x, edge_index, batch, W_l, W_r, att, b_conv, ln_w, ln_b,
           W_out, b_out):
    n_edges = edge_index.shape[1]
    xlr = pl.pallas_call(
        _mm_body,
        out_shape=jax.ShapeDtypeStruct((N_NODES, F_IN), jnp.float32),
    )(x, W_l, W_r)
    pk2 = (edge_index[0] | (edge_index[1] << 16)).reshape(NC * NS, -1)
    seg_split, segw = 64, 65 * CHUNK
    seg0 = pk2[:, :segw]
    seg1 = pk2[:, seg_split * CHUNK:]
    seg1 = jnp.pad(seg1, ((0, 0), (0, segw - seg1.shape[1])))
    pk3 = jnp.stack([seg0, seg1], axis=1)
    parts = _edge_pass(n_edges)(xlr, pk3, att)
    y = pl.pallas_call(
        _post_body,
        out_shape=jax.ShapeDtypeStruct((N_GRAPHS, 1), jnp.float32),
    )(parts, b_conv.reshape(1, HID), ln_w.reshape(1, HID),
      ln_b.reshape(1, HID), batch.reshape(N_NODES, 1), W_out,
      b_out.reshape(1, 1))
    return y


# speedup vs baseline: 1.1100x; 1.1100x over previous
import functools

import jax
import jax.numpy as jnp
from jax import lax
from jax.experimental import pallas as pl
from jax.experimental.pallas import tpu as pltpu
from jax.experimental.pallas import tpu_sc as plsc

N_NODES = 10000
N_PAD = 10240
F_IN = 128
HID = 64
N_GRAPHS = 16

ROW = 128
CHUNK = 80
NC = 2
NS = 16
LANES = 16


def _mm_body(x_ref, wl_ref, wr_ref, xlr_ref):
    x = x_ref[...]
    xlr_ref[:, :HID] = jnp.dot(x, wl_ref[...],
                               preferred_element_type=jnp.float32)
    xlr_ref[:, HID:] = jnp.dot(x, wr_ref[...],
                               preferred_element_type=jnp.float32)


def _edge_pass(n_edges):
    e_per_tile = n_edges // (NC * NS)
    n_chunks = e_per_tile // CHUNK
    assert e_per_tile * NC * NS == n_edges and n_chunks * CHUNK == e_per_tile
    rows_per_tile = N_PAD // NS
    zrows = 32
    mesh = plsc.VectorSubcoreMesh(core_axis_name="c", subcore_axis_name="s")

    seg_split = 64
    segw = (seg_split + 1) * CHUNK

    @functools.partial(
        pl.kernel,
        out_type=jax.ShapeDtypeStruct((NC, N_PAD, ROW), jnp.float32),
        mesh=mesh,
        scratch_types=[
            pltpu.VMEM((segw,), jnp.int32),
            pltpu.VMEM((CHUNK,), jnp.int32),
            pltpu.VMEM((CHUNK,), jnp.int32),
            pltpu.VMEM((CHUNK,), jnp.int32),
            pltpu.VMEM((CHUNK,), jnp.int32),
            pltpu.VMEM((CHUNK, F_IN), jnp.float32),
            pltpu.VMEM((CHUNK, F_IN), jnp.float32),
            pltpu.VMEM((CHUNK, F_IN), jnp.float32),
            pltpu.VMEM((CHUNK, ROW), jnp.float32),
            pltpu.VMEM((HID,), jnp.float32),
            pltpu.VMEM((LANES * 17,), jnp.float32),
            pltpu.VMEM((CHUNK,), jnp.float32),
            pltpu.VMEM_SHARED((N_PAD, ROW), jnp.float32),
            pltpu.SemaphoreType.DMA,
            pltpu.SemaphoreType.DMA,
            pltpu.SemaphoreType.DMA,
        ],
        compiler_params=pltpu.CompilerParams(needs_layout_passes=False),
    )
    def k(xlr, pk3, att, out,
          idx_pk, sr0, sr1, dr0, dr1, rs0, rs1, rd, obuf, attv, qbuf,
          wbuf, acc, sgs0, sgs1, sgd):
        c = lax.axis_index("c")
        s = lax.axis_index("s")
        tid = c * NS + s

        def zrow(i, carry):
            for j in range(ROW // LANES):
                obuf[i, pl.ds(LANES * j, LANES)] = jnp.zeros((LANES,),
                                                             jnp.float32)
            return carry
        lax.fori_loop(0, CHUNK, zrow, 0)
        for t in range(rows_per_tile // CHUNK):
            row0 = s * rows_per_tile + t * CHUNK
            pltpu.sync_copy(obuf, acc.at[pl.ds(row0, CHUNK)])

        pltpu.sync_copy(att, attv)
        pltpu.sync_copy(pk3.at[tid, 0], idx_pk)
        plsc.subcore_barrier()

        iota = lax.iota(jnp.int32, LANES)
        zero_i = jnp.zeros((LANES,), jnp.int32)
        att_vs = [attv[pl.ds(LANES * kk, LANES)] for kk in range(HID // LANES)]

        def unpack(lch, idx_sr, idx_dr):
            for g in range(CHUNK // LANES):
                pk = idx_pk[pl.ds(lch * CHUNK + g * LANES, LANES)]
                idx_sr[pl.ds(g * LANES, LANES)] = pk & 0xFFFF
                idx_dr[pl.ds(g * LANES, LANES)] = pk >> 16

        def phase1(rs):
            def group_body(g, carry):
                svec = jnp.zeros((LANES,), jnp.float32)
                for e16 in range(LANES):
                    e = g * LANES + e16
                    ps = []
                    for kk in range(HID // LANES):
                        l = rs[e, pl.ds(LANES * kk, LANES)]
                        r = rd[e, pl.ds(HID + LANES * kk, LANES)]
                        v = l + r
                        v = jnp.maximum(v, 0.2 * v)
                        ps.append(v * att_vs[kk])
                    s_e = jnp.sum((ps[0] + ps[1]) + (ps[2] + ps[3]))
                    svec = jnp.where(iota == e16, s_e, svec)
                wbuf[pl.ds(g * LANES, LANES)] = jnp.exp(svec)
                return carry
            lax.fori_loop(0, CHUNK // LANES, group_body, 0)

        def phase2(rs):
            def edge_body(e, carry):
                wv = plsc.load_gather(wbuf, [zero_i + e])
                for k2 in range(HID // LANES):
                    obuf[e, pl.ds(LANES * k2, LANES)] = (
                        wv * rs[e, pl.ds(LANES * k2, LANES)])
                obuf[e, pl.ds(HID, LANES)] = jnp.where(iota == 0, wv, 0.0)
                return carry
            lax.fori_loop(0, CHUNK, edge_body, 0, unroll=8)

        def fire_rd(idx_dr):
            pltpu.async_copy(xlr.at[idx_dr], rd, sgd)

        def wait_rd(idx_dr):
            pltpu.make_async_copy(xlr.at[idx_dr], rd, sgd).wait()

        def fire_rs(idx_sr, rs, sem):
            pltpu.async_copy(xlr.at[idx_sr], rs, sem)

        def wait_rs(idx_sr, rs, sem):
            pltpu.make_async_copy(xlr.at[idx_sr], rs, sem).wait()

        def section(lch_next, sr_cur, dr_cur, rs_cur, sem_cur,
                    sr_nxt, dr_nxt, rs_nxt, sem_nxt, last=False):
            if not last:
                unpack(lch_next, sr_nxt, dr_nxt)
            wait_rd(dr_cur)
            wait_rs(sr_cur, rs_cur, sem_cur)
            phase1(rs_cur)
            if not last:
                fire_rd(dr_nxt)
            phase2(rs_cur)
            if not last:
                fire_rs(sr_nxt, rs_nxt, sem_nxt)
            pltpu.sync_copy(obuf, acc.at[dr_cur], add=True)

        def pair_body(i, carry):
            section(2 * i + 1, sr0, dr0, rs0, sgs0, sr1, dr1, rs1, sgs1)
            section(2 * i + 2, sr1, dr1, rs1, sgs1, sr0, dr0, rs0, sgs0)
            return carry

        unpack(0, sr0, dr0)
        fire_rd(dr0)
        fire_rs(sr0, rs0, sgs0)
        lax.fori_loop(0, seg_split // 2, pair_body, 0)

        pltpu.sync_copy(pk3.at[tid, 1], idx_pk)
        lax.fori_loop(0, (n_chunks - seg_split - 1) // 2, pair_body, 0)
        section(0, sr0, dr0, rs0, sgs0, sr1, dr1, rs1, sgs1, last=True)

        plsc.subcore_barrier()
        for t in range(rows_per_tile // CHUNK):
            row0 = s * rows_per_tile + t * CHUNK
            pltpu.sync_copy(acc.at[pl.ds(row0, CHUNK)], obuf)
            pltpu.sync_copy(obuf, out.at[c, pl.ds(row0, CHUNK)])

    return k


def _post_body(parts_ref, bconv_ref, lnw_ref, lnb_ref, batch_ref,
               wout_ref, bout_ref, y_ref):
    accp = parts_ref[0, :N_NODES] + parts_ref[1, :N_NODES]
    num = accp[:, :HID]
    den = accp[:, HID:HID + 1]
    h = jnp.maximum(num / (den + 1e-16) + bconv_ref[...], 0.0)
    mu = jnp.mean(h)
    var = jnp.mean((h - mu) ** 2)
    hn = (h - mu) / (jnp.sqrt(var) + 1e-5) * lnw_ref[...] + lnb_ref[...]
    onehot = (batch_ref[...] == lax.broadcasted_iota(
        jnp.int32, (N_NODES, N_GRAPHS), 1)).astype(jnp.float32)
    sums = lax.dot_general(onehot, hn, (((0,), (0,)), ((), ())),
                           preferred_element_type=jnp.float32)
    cnts = jnp.sum(onehot, axis=0)
    pooled = sums / jnp.maximum(cnts, 1.0)[:, None]
    y = jnp.dot(pooled, wout_ref[...], preferred_element_type=jnp.float32)
    y_ref[...] = jax.nn.sigmoid(y + bout_ref[...])


def kernel(x, edge_index, batch, W_l, W_r, att, b_conv, ln_w, ln_b,
           W_out, b_out):
    n_edges = edge_index.shape[1]
    xlr = pl.pallas_call(
        _mm_body,
        out_shape=jax.ShapeDtypeStruct((N_NODES, F_IN), jnp.float32),
    )(x, W_l, W_r)
    pk2 = (edge_index[0] | (edge_index[1] << 16)).reshape(NC * NS, -1)
    seg_split, segw = 64, 65 * CHUNK
    seg0 = pk2[:, :segw]
    seg1 = pk2[:, seg_split * CHUNK:]
    seg1 = jnp.pad(seg1, ((0, 0), (0, segw - seg1.shape[1])))
    pk3 = jnp.stack([seg0, seg1], axis=1)
    parts = _edge_pass(n_edges)(xlr, pk3, att)
    y = pl.pallas_call(
        _post_body,
        out_shape=jax.ShapeDtypeStruct((N_GRAPHS, 1), jnp.float32),
    )(parts, b_conv.reshape(1, HID), ln_w.reshape(1, HID),
      ln_b.reshape(1, HID), batch.reshape(N_NODES, 1), W_out,
      b_out.reshape(1, 1))
    return y


# speedup vs baseline: 1.2025x; 1.0833x over previous
import functools

import jax
import jax.numpy as jnp
from jax import lax
from jax.experimental import pallas as pl
from jax.experimental.pallas import tpu as pltpu
from jax.experimental.pallas import tpu_sc as plsc

N_NODES = 10000
N_PAD = 10240
F_IN = 128
HID = 64
N_GRAPHS = 16

ROW = 128
CHUNK = 80
NC = 2
NS = 16
LANES = 16


def _mm_body(x_ref, wl_ref, wr_ref, xlr_ref):
    x = x_ref[...]
    xlr_ref[:, :HID] = jnp.dot(x, wl_ref[...],
                               preferred_element_type=jnp.float32)
    xlr_ref[:, HID:] = jnp.dot(x, wr_ref[...],
                               preferred_element_type=jnp.float32)


def _edge_pass(n_edges):
    e_per_tile = n_edges // (NC * NS)
    n_chunks = e_per_tile // CHUNK
    assert e_per_tile * NC * NS == n_edges and n_chunks * CHUNK == e_per_tile
    rows_per_tile = N_PAD // NS
    zrows = 32
    mesh = plsc.VectorSubcoreMesh(core_axis_name="c", subcore_axis_name="s")

    seg_split = 64
    segw = (seg_split + 1) * CHUNK

    @functools.partial(
        pl.kernel,
        out_type=jax.ShapeDtypeStruct((NC, N_PAD, ROW), jnp.float32),
        mesh=mesh,
        scratch_types=[
            pltpu.VMEM((segw,), jnp.int32),
            pltpu.VMEM((CHUNK,), jnp.int32),
            pltpu.VMEM((CHUNK,), jnp.int32),
            pltpu.VMEM((CHUNK,), jnp.int32),
            pltpu.VMEM((CHUNK,), jnp.int32),
            pltpu.VMEM((CHUNK, F_IN), jnp.float32),
            pltpu.VMEM((CHUNK, F_IN), jnp.float32),
            pltpu.VMEM((CHUNK, F_IN), jnp.float32),
            pltpu.VMEM((CHUNK, ROW), jnp.float32),
            pltpu.VMEM((HID,), jnp.float32),
            pltpu.VMEM((LANES * 17,), jnp.float32),
            pltpu.VMEM((CHUNK,), jnp.float32),
            pltpu.VMEM_SHARED((N_PAD, ROW), jnp.float32),
            pltpu.SemaphoreType.DMA,
            pltpu.SemaphoreType.DMA,
            pltpu.SemaphoreType.DMA,
        ],
        compiler_params=pltpu.CompilerParams(needs_layout_passes=False),
    )
    def k(xlr, pk3, att, out,
          idx_pk, sr0, sr1, dr0, dr1, rs0, rs1, rd, obuf, attv, qbuf,
          wbuf, acc, sgs0, sgs1, sgd):
        c = lax.axis_index("c")
        s = lax.axis_index("s")
        tid = c * NS + s

        def zrow(i, carry):
            for j in range(ROW // LANES):
                obuf[i, pl.ds(LANES * j, LANES)] = jnp.zeros((LANES,),
                                                             jnp.float32)
            return carry
        lax.fori_loop(0, CHUNK, zrow, 0)
        for t in range(rows_per_tile // CHUNK):
            row0 = s * rows_per_tile + t * CHUNK
            pltpu.sync_copy(obuf, acc.at[pl.ds(row0, CHUNK)])

        pltpu.sync_copy(att, attv)
        pltpu.sync_copy(pk3.at[tid, 0], idx_pk)
        plsc.subcore_barrier()

        iota = lax.iota(jnp.int32, LANES)
        zero_i = jnp.zeros((LANES,), jnp.int32)
        att_vs = [attv[pl.ds(LANES * kk, LANES)] for kk in range(HID // LANES)]

        def unpack(lch, idx_sr, idx_dr):
            for g in range(CHUNK // LANES):
                pk = idx_pk[pl.ds(lch * CHUNK + g * LANES, LANES)]
                idx_sr[pl.ds(g * LANES, LANES)] = pk & 0xFFFF
                idx_dr[pl.ds(g * LANES, LANES)] = pk >> 16

        bit_masks = [(iota & (1 << b)) == 0 for b in range(4)]

        def phase1(rs):
            def group_body(g, carry):
                svals = []
                for e16 in range(LANES):
                    e = g * LANES + e16
                    ps = []
                    for kk in range(HID // LANES):
                        l = rs[e, pl.ds(LANES * kk, LANES)]
                        r = rd[e, pl.ds(HID + LANES * kk, LANES)]
                        v = l + r
                        v = jnp.maximum(v, 0.2 * v)
                        ps.append(v * att_vs[kk])
                    s_e = jnp.sum((ps[0] + ps[1]) + (ps[2] + ps[3]))
                    svals.append(jnp.full((LANES,), s_e))
                for b in range(4):
                    svals = [jnp.where(bit_masks[b], svals[2 * i],
                                       svals[2 * i + 1])
                             for i in range(len(svals) // 2)]
                wbuf[pl.ds(g * LANES, LANES)] = jnp.exp(svals[0])
                return carry
            lax.fori_loop(0, CHUNK // LANES, group_body, 0)

        def phase2(rs):
            def edge_body(e, carry):
                wv = plsc.load_gather(wbuf, [zero_i + e])
                for k2 in range(HID // LANES):
                    obuf[e, pl.ds(LANES * k2, LANES)] = (
                        wv * rs[e, pl.ds(LANES * k2, LANES)])
                obuf[e, pl.ds(HID, LANES)] = jnp.where(iota == 0, wv, 0.0)
                return carry
            lax.fori_loop(0, CHUNK, edge_body, 0, unroll=4)

        def fire_rd(idx_dr):
            pltpu.async_copy(xlr.at[idx_dr], rd, sgd)

        def wait_rd(idx_dr):
            pltpu.make_async_copy(xlr.at[idx_dr], rd, sgd).wait()

        def fire_rs(idx_sr, rs, sem):
            pltpu.async_copy(xlr.at[idx_sr], rs, sem)

        def wait_rs(idx_sr, rs, sem):
            pltpu.make_async_copy(xlr.at[idx_sr], rs, sem).wait()

        def section(lch_next, sr_cur, dr_cur, rs_cur, sem_cur,
                    sr_nxt, dr_nxt, rs_nxt, sem_nxt, last=False):
            if not last:
                unpack(lch_next, sr_nxt, dr_nxt)
            wait_rd(dr_cur)
            wait_rs(sr_cur, rs_cur, sem_cur)
            phase1(rs_cur)
            if not last:
                fire_rd(dr_nxt)
            phase2(rs_cur)
            if not last:
                fire_rs(sr_nxt, rs_nxt, sem_nxt)
            pltpu.sync_copy(obuf, acc.at[dr_cur], add=True)

        def pair_body(i, carry):
            section(2 * i + 1, sr0, dr0, rs0, sgs0, sr1, dr1, rs1, sgs1)
            section(2 * i + 2, sr1, dr1, rs1, sgs1, sr0, dr0, rs0, sgs0)
            return carry

        unpack(0, sr0, dr0)
        fire_rd(dr0)
        fire_rs(sr0, rs0, sgs0)
        lax.fori_loop(0, seg_split // 2, pair_body, 0)

        pltpu.sync_copy(pk3.at[tid, 1], idx_pk)
        lax.fori_loop(0, (n_chunks - seg_split - 1) // 2, pair_body, 0)
        section(0, sr0, dr0, rs0, sgs0, sr1, dr1, rs1, sgs1, last=True)

        plsc.subcore_barrier()
        stage = [(rs0, sgs0), (rs1, sgs1), (rd, sgd)]
        n_rb = rows_per_tile // CHUNK
        for t in range(n_rb):
            buf, sem = stage[t % 3]
            row0 = s * rows_per_tile + t * CHUNK
            pltpu.sync_copy(acc.at[pl.ds(row0, CHUNK)], buf)
            if t >= 3:
                pbuf, psem = stage[(t - 3) % 3]
                prow = s * rows_per_tile + (t - 3) * CHUNK
                pltpu.make_async_copy(pbuf,
                                      out.at[c, pl.ds(prow, CHUNK)],
                                      psem).wait()
            pltpu.async_copy(buf, out.at[c, pl.ds(row0, CHUNK)],
                             sem)
        for t in range(max(n_rb - 3, 0), n_rb):
            buf, sem = stage[t % 3]
            row0 = s * rows_per_tile + t * CHUNK
            pltpu.make_async_copy(buf,
                                  out.at[c, pl.ds(row0, CHUNK)], sem).wait()

    return k


def _post_body(parts_ref, bconv_ref, lnw_ref, lnb_ref, batch_ref,
               wout_ref, bout_ref, y_ref):
    accp = parts_ref[0, :N_NODES] + parts_ref[1, :N_NODES]
    num = accp[:, :HID]
    den = accp[:, HID:HID + 1]
    h = jnp.maximum(num / (den + 1e-16) + bconv_ref[...], 0.0)
    mu = jnp.mean(h)
    var = jnp.mean((h - mu) ** 2)
    hn = (h - mu) / (jnp.sqrt(var) + 1e-5) * lnw_ref[...] + lnb_ref[...]
    onehot = (batch_ref[...] == lax.broadcasted_iota(
        jnp.int32, (N_NODES, N_GRAPHS), 1)).astype(jnp.float32)
    sums = lax.dot_general(onehot, hn, (((0,), (0,)), ((), ())),
                           preferred_element_type=jnp.float32)
    cnts = jnp.sum(onehot, axis=0)
    pooled = sums / jnp.maximum(cnts, 1.0)[:, None]
    y = jnp.dot(pooled, wout_ref[...], preferred_element_type=jnp.float32)
    y_ref[...] = jax.nn.sigmoid(y + bout_ref[...])


def kernel(x, edge_index, batch, W_l, W_r, att, b_conv, ln_w, ln_b,
           W_out, b_out):
    n_edges = edge_index.shape[1]
    xlr = pl.pallas_call(
        _mm_body,
        out_shape=jax.ShapeDtypeStruct((N_NODES, F_IN), jnp.float32),
    )(x, W_l, W_r)
    pk2 = (edge_index[0] | (edge_index[1] << 16)).reshape(NC * NS, -1)
    seg_split, segw = 64, 65 * CHUNK
    seg0 = pk2[:, :segw]
    seg1 = pk2[:, seg_split * CHUNK:]
    seg1 = jnp.pad(seg1, ((0, 0), (0, segw - seg1.shape[1])))
    pk3 = jnp.stack([seg0, seg1], axis=1)
    parts = _edge_pass(n_edges)(xlr, pk3, att)
    y = pl.pallas_call(
        _post_body,
        out_shape=jax.ShapeDtypeStruct((N_GRAPHS, 1), jnp.float32),
    )(parts, b_conv.reshape(1, HID), ln_w.reshape(1, HID),
      ln_b.reshape(1, HID), batch.reshape(N_NODES, 1), W_out,
      b_out.reshape(1, 1))
    return y


# speedup vs baseline: 1.4274x; 1.1870x over previous
import functools

import jax
import jax.numpy as jnp
from jax import lax
from jax.experimental import pallas as pl
from jax.experimental.pallas import tpu as pltpu
from jax.experimental.pallas import tpu_sc as plsc

N_NODES = 10000
N_PAD = 10240
F_IN = 128
HID = 64
N_GRAPHS = 16

ROW = 128
CHUNK = 80
NC = 2
NS = 16
LANES = 16


def _mm_body(x_ref, wl_ref, wr_ref, xlr_ref):
    x = x_ref[...]
    xlr_ref[:, :HID] = jnp.dot(x, wl_ref[...],
                               preferred_element_type=jnp.float32)
    xlr_ref[:, HID:] = jnp.dot(x, wr_ref[...],
                               preferred_element_type=jnp.float32)


def _edge_pass(n_edges):
    e_per_tile = n_edges // (NC * NS)
    n_chunks = e_per_tile // CHUNK
    assert e_per_tile * NC * NS == n_edges and n_chunks * CHUNK == e_per_tile
    rows_per_tile = N_PAD // NS
    zrows = 32
    mesh = plsc.VectorSubcoreMesh(core_axis_name="c", subcore_axis_name="s")

    seg_split = 64
    segw = (seg_split + 1) * CHUNK

    @functools.partial(
        pl.kernel,
        out_type=jax.ShapeDtypeStruct((NC, N_PAD, ROW), jnp.float32),
        mesh=mesh,
        scratch_types=[
            pltpu.VMEM((segw,), jnp.int32),
            pltpu.VMEM((CHUNK,), jnp.int32),
            pltpu.VMEM((CHUNK,), jnp.int32),
            pltpu.VMEM((CHUNK,), jnp.int32),
            pltpu.VMEM((CHUNK,), jnp.int32),
            pltpu.VMEM((CHUNK, F_IN), jnp.float32),
            pltpu.VMEM((CHUNK, F_IN), jnp.float32),
            pltpu.VMEM((CHUNK, F_IN), jnp.float32),
            pltpu.VMEM((CHUNK, ROW), jnp.float32),
            pltpu.VMEM((HID,), jnp.float32),
            pltpu.VMEM((LANES * 17,), jnp.float32),
            pltpu.VMEM((CHUNK,), jnp.float32),
            pltpu.VMEM_SHARED((N_PAD, ROW), jnp.float32),
            pltpu.SemaphoreType.DMA,
            pltpu.SemaphoreType.DMA,
            pltpu.SemaphoreType.DMA,
        ],
        compiler_params=pltpu.CompilerParams(needs_layout_passes=False),
    )
    def k(xlr, pk3, att, out,
          idx_pk, sr0, sr1, dr0, dr1, rs0, rs1, rd, obuf, attv, qbuf,
          wbuf, acc, sgs0, sgs1, sgd):
        c = lax.axis_index("c")
        s = lax.axis_index("s")
        tid = c * NS + s

        def zrow(i, carry):
            for j in range(ROW // LANES):
                obuf[i, pl.ds(LANES * j, LANES)] = jnp.zeros((LANES,),
                                                             jnp.float32)
            return carry
        lax.fori_loop(0, CHUNK, zrow, 0)
        for t in range(rows_per_tile // CHUNK):
            row0 = s * rows_per_tile + t * CHUNK
            pltpu.sync_copy(obuf, acc.at[pl.ds(row0, CHUNK)])

        pltpu.sync_copy(att, attv)
        pltpu.sync_copy(pk3.at[tid, 0], idx_pk)
        plsc.subcore_barrier()

        iota = lax.iota(jnp.int32, LANES)
        zero_i = jnp.zeros((LANES,), jnp.int32)
        att_vs = [attv[pl.ds(LANES * kk, LANES)] for kk in range(HID // LANES)]

        def unpack(lch, idx_sr, idx_dr):
            for g in range(CHUNK // LANES):
                pk = idx_pk[pl.ds(lch * CHUNK + g * LANES, LANES)]
                idx_sr[pl.ds(g * LANES, LANES)] = pk & 0xFFFF
                idx_dr[pl.ds(g * LANES, LANES)] = pk >> 16

        bit_masks = [(iota & (1 << b)) == 0 for b in range(4)]

        def phase1(rs):
            def group_body(g, carry):
                svals = []
                for e16 in range(LANES):
                    e = g * LANES + e16
                    ps = []
                    for kk in range(HID // LANES):
                        l = rs[e, pl.ds(LANES * kk, LANES)]
                        r = rd[e, pl.ds(HID + LANES * kk, LANES)]
                        v = l + r
                        v = jnp.maximum(v, 0.2 * v)
                        ps.append(v * att_vs[kk])
                    s_e = jnp.sum((ps[0] + ps[1]) + (ps[2] + ps[3]))
                    svals.append(jnp.full((LANES,), s_e))
                for b in range(4):
                    svals = [jnp.where(bit_masks[b], svals[2 * i],
                                       svals[2 * i + 1])
                             for i in range(len(svals) // 2)]
                wbuf[pl.ds(g * LANES, LANES)] = jnp.exp(svals[0])
                return carry
            lax.fori_loop(0, CHUNK // LANES, group_body, 0)

        def phase2(rs):
            def edge_body(e, carry):
                wv = plsc.load_gather(wbuf, [zero_i + e])
                for k2 in range(HID // LANES):
                    obuf[e, pl.ds(LANES * k2, LANES)] = (
                        wv * rs[e, pl.ds(LANES * k2, LANES)])
                obuf[e, pl.ds(HID, LANES)] = jnp.where(iota == 0, wv, 0.0)
                return carry
            lax.fori_loop(0, CHUNK, edge_body, 0, unroll=4)

        def fire_rd(idx_dr):
            pltpu.async_copy(xlr.at[idx_dr], rd, sgd)

        def wait_rd(idx_dr):
            pltpu.make_async_copy(xlr.at[idx_dr], rd, sgd).wait()

        def fire_rs(idx_sr, rs, sem):
            pltpu.async_copy(xlr.at[idx_sr], rs, sem)

        def wait_rs(idx_sr, rs, sem):
            pltpu.make_async_copy(xlr.at[idx_sr], rs, sem).wait()

        def section(lch_next, sr_cur, dr_cur, rs_cur, sem_cur,
                    sr_nxt, dr_nxt, rs_nxt, sem_nxt, last=False):
            if not last:
                unpack(lch_next, sr_nxt, dr_nxt)
                fire_rs(sr_nxt, rs_nxt, sem_nxt)
            wait_rd(dr_cur)
            wait_rs(sr_cur, rs_cur, sem_cur)
            phase1(rs_cur)
            if not last:
                fire_rd(dr_nxt)
            phase2(rs_cur)
            pltpu.sync_copy(obuf, acc.at[dr_cur], add=True)

        def pair_body(i, carry):
            section(2 * i + 1, sr0, dr0, rs0, sgs0, sr1, dr1, rs1, sgs1)
            section(2 * i + 2, sr1, dr1, rs1, sgs1, sr0, dr0, rs0, sgs0)
            return carry

        unpack(0, sr0, dr0)
        fire_rd(dr0)
        fire_rs(sr0, rs0, sgs0)
        lax.fori_loop(0, seg_split // 2, pair_body, 0)

        pltpu.sync_copy(pk3.at[tid, 1], idx_pk)
        lax.fori_loop(0, (n_chunks - seg_split - 1) // 2, pair_body, 0)
        section(0, sr0, dr0, rs0, sgs0, sr1, dr1, rs1, sgs1, last=True)

        plsc.subcore_barrier()
        stage = [(rs0, sgs0), (rs1, sgs1), (rd, sgd)]
        n_rb = rows_per_tile // CHUNK
        for t in range(n_rb):
            buf, sem = stage[t % 3]
            row0 = s * rows_per_tile + t * CHUNK
            pltpu.sync_copy(acc.at[pl.ds(row0, CHUNK)], buf)
            if t >= 3:
                pbuf, psem = stage[(t - 3) % 3]
                prow = s * rows_per_tile + (t - 3) * CHUNK
                pltpu.make_async_copy(pbuf,
                                      out.at[c, pl.ds(prow, CHUNK)],
                                      psem).wait()
            pltpu.async_copy(buf, out.at[c, pl.ds(row0, CHUNK)],
                             sem)
        for t in range(max(n_rb - 3, 0), n_rb):
            buf, sem = stage[t % 3]
            row0 = s * rows_per_tile + t * CHUNK
            pltpu.make_async_copy(buf,
                                  out.at[c, pl.ds(row0, CHUNK)], sem).wait()

    return k


def _post_body(parts_ref, bconv_ref, lnw_ref, lnb_ref, batch_ref,
               wout_ref, bout_ref, y_ref):
    accp = parts_ref[0, :N_NODES] + parts_ref[1, :N_NODES]
    num = accp[:, :HID]
    den = accp[:, HID:HID + 1]
    h = jnp.maximum(num / (den + 1e-16) + bconv_ref[...], 0.0)
    mu = jnp.mean(h)
    var = jnp.mean((h - mu) ** 2)
    hn = (h - mu) / (jnp.sqrt(var) + 1e-5) * lnw_ref[...] + lnb_ref[...]
    onehot = (batch_ref[...] == lax.broadcasted_iota(
        jnp.int32, (N_NODES, N_GRAPHS), 1)).astype(jnp.float32)
    sums = lax.dot_general(onehot, hn, (((0,), (0,)), ((), ())),
                           preferred_element_type=jnp.float32)
    cnts = jnp.sum(onehot, axis=0)
    pooled = sums / jnp.maximum(cnts, 1.0)[:, None]
    y = jnp.dot(pooled, wout_ref[...], preferred_element_type=jnp.float32)
    y_ref[...] = jax.nn.sigmoid(y + bout_ref[...])


def kernel(x, edge_index, batch, W_l, W_r, att, b_conv, ln_w, ln_b,
           W_out, b_out):
    n_edges = edge_index.shape[1]
    xlr = pl.pallas_call(
        _mm_body,
        out_shape=jax.ShapeDtypeStruct((N_NODES, F_IN), jnp.float32),
    )(x, W_l, W_r)
    pk2 = (edge_index[0] | (edge_index[1] << 16)).reshape(NC * NS, -1)
    seg_split, segw = 64, 65 * CHUNK
    seg0 = pk2[:, :segw]
    seg1 = pk2[:, seg_split * CHUNK:]
    seg1 = jnp.pad(seg1, ((0, 0), (0, segw - seg1.shape[1])))
    pk3 = jnp.stack([seg0, seg1], axis=1)
    parts = _edge_pass(n_edges)(xlr, pk3, att)
    y = pl.pallas_call(
        _post_body,
        out_shape=jax.ShapeDtypeStruct((N_GRAPHS, 1), jnp.float32),
    )(parts, b_conv.reshape(1, HID), ln_w.reshape(1, HID),
      ln_b.reshape(1, HID), batch.reshape(N_NODES, 1), W_out,
      b_out.reshape(1, 1))
    return y
